# Initial kernel scaffold; baseline (speedup 1.0000x reference)
import jax
import jax.numpy as jnp
from jax.experimental import pallas as pl


def kernel(x, edge_index, W_self, W_neigh, b):
    raise NotImplementedError("write your pallas kernel here")


# speedup vs baseline: 8.2667x; 8.2667x over previous
import functools

import jax
import jax.numpy as jnp
from jax import lax
from jax.experimental import pallas as pl
from jax.experimental.pallas import tpu as pltpu
from jax.experimental.pallas import tpu_sc as plsc

N = 10000
E = 320000
D = 128
NC = 2
NS = 16
NW = NC * NS
C = 80
EPW = E // NW
NCHUNK = EPW // C
IB = 25
NOB = NCHUNK // IB
NP = 10240
RPT = NP // NS
DGC = 16
TB = 2000

_mesh = plsc.VectorSubcoreMesh(core_axis_name="c", subcore_axis_name="s")


@functools.partial(
    pl.kernel,
    out_type=(
        jax.ShapeDtypeStruct((NC, NP, D), jnp.float32),
        jax.ShapeDtypeStruct((NC, NP, DGC), jnp.float32),
    ),
    mesh=_mesh,
    compiler_params=pltpu.CompilerParams(use_tc_tiling_on_sc=False),
    scratch_types=[
        pltpu.VMEM((IB, C), jnp.int32),
        pltpu.VMEM((IB, C), jnp.int32),
        pltpu.VMEM((C, D), jnp.float32),
        pltpu.VMEM((C, DGC), jnp.float32),
        pltpu.VMEM((160, DGC), jnp.float32),
        pltpu.VMEM_SHARED((NP, D), jnp.float32),
        pltpu.VMEM_SHARED((NP, DGC), jnp.float32),
        pltpu.SemaphoreType.DMA,
    ],
)
def _sc_segsum(src_hbm, dst_hbm, x_hbm, sum_out, deg_out,
               src_v, dst_v, rows_v, ones_v, zdeg_v,
               acc_sh, dgacc_sh, sem):
    cid = lax.axis_index("c")
    sid = lax.axis_index("s")
    zero16 = jnp.zeros((16,), jnp.float32)
    one16 = jnp.ones((16,), jnp.float32)

    def fill_ones(i, carry):
        ones_v[i, :] = one16
        return carry

    lax.fori_loop(0, C, fill_ones, 0)

    def zrow(i, carry):
        rows_v[i // 8, pl.ds((i % 8) * 16, 16)] = zero16
        return carry

    lax.fori_loop(0, C * 8, zrow, 0)

    def zdeg(i, carry):
        zdeg_v[i, :] = zero16
        return carry

    lax.fori_loop(0, 160, zdeg, 0)

    for k in range(RPT // C):
        pltpu.sync_copy(rows_v, acc_sh.at[pl.ds(sid * RPT + k * C, C)])
    for k in range(RPT // 160):
        pltpu.sync_copy(zdeg_v, dgacc_sh.at[pl.ds(sid * RPT + k * 160, 160)])
    plsc.subcore_barrier()

    def outer(ob, carry):
        pltpu.sync_copy(src_hbm.at[cid, sid, ob], src_v)
        pltpu.sync_copy(dst_hbm.at[cid, sid, ob], dst_v)

        def body(j, c2):
            pltpu.async_copy(x_hbm.at[src_v.at[j]], rows_v, sem).wait()
            pltpu.sync_copy(rows_v, acc_sh.at[dst_v.at[j]], add=True)
            pltpu.sync_copy(ones_v, dgacc_sh.at[dst_v.at[j]], add=True)
            return c2

        lax.fori_loop(0, IB, body, 0)
        return carry

    lax.fori_loop(0, NOB, outer, 0)

    plsc.subcore_barrier()

    pltpu.sync_copy(acc_sh.at[pl.ds(sid * RPT, RPT)],
                    sum_out.at[cid].at[pl.ds(sid * RPT, RPT)])
    pltpu.sync_copy(dgacc_sh.at[pl.ds(sid * RPT, RPT)],
                    deg_out.at[cid].at[pl.ds(sid * RPT, RPT)])


def _tc_combine(x_ref, p_ref, dg_ref, ws_ref, wn_ref, b_ref, o_ref):
    p = p_ref[0] + p_ref[1]
    deg = jnp.sum(dg_ref[...], axis=(0, 2)) * (1.0 / DGC)
    inv = 1.0 / jnp.maximum(deg, 1.0)
    h = p * inv[:, None]
    o_ref[...] = (
        jnp.dot(x_ref[...], ws_ref[...], preferred_element_type=jnp.float32)
        + jnp.dot(h, wn_ref[...], preferred_element_type=jnp.float32)
        + b_ref[0][None, :]
    )


@jax.jit
def kernel(x, edge_index, W_self, W_neigh, b):
    src = edge_index[0].reshape(NC, NS, NOB, IB, C)
    dst = edge_index[1].reshape(NC, NS, NOB, IB, C)
    sum_p, deg_p = _sc_segsum(src, dst, x)
    b8 = jnp.broadcast_to(b.astype(jnp.float32), (8, D))
    out = pl.pallas_call(
        _tc_combine,
        grid=(N // TB,),
        in_specs=[
            pl.BlockSpec((TB, D), lambda i: (i, 0)),
            pl.BlockSpec((NC, TB, D), lambda i: (0, i, 0)),
            pl.BlockSpec((NC, TB, DGC), lambda i: (0, i, 0)),
            pl.BlockSpec((D, D), lambda i: (0, 0)),
            pl.BlockSpec((D, D), lambda i: (0, 0)),
            pl.BlockSpec((8, D), lambda i: (0, 0)),
        ],
        out_specs=pl.BlockSpec((TB, D), lambda i: (i, 0)),
        out_shape=jax.ShapeDtypeStruct((N, D), jnp.float32),
    )(x, sum_p, deg_p, W_self, W_neigh, b8)
    return out
